# baseline (speedup 1.0000x reference)
import functools

import jax
import jax.numpy as jnp
from jax import lax
from jax.experimental import pallas as pl
from jax.experimental.pallas import tpu as pltpu
from jax.experimental.pallas import tpu_sc as plsc

VOCAB = 1_000_000
B = 1024
S = 200
D = 64
MD = 1024
N = B * S
NW = 32
PER_W = N // NW
CHUNK = 128
NCH = PER_W // CHUNK
LANES = 16

_mesh = plsc.VectorSubcoreMesh(core_axis_name="c", subcore_axis_name="s")


@functools.partial(
    pl.kernel,
    mesh=_mesh,
    compiler_params=pltpu.CompilerParams(use_tc_tiling_on_sc=True),
    out_type=jax.ShapeDtypeStruct((N, 2 * D), jnp.float32),
    scratch_types=[
        pltpu.VMEM((NCH, CHUNK), jnp.int32),
        pltpu.VMEM((NCH, CHUNK), jnp.int32),
        pltpu.VMEM((NCH, CHUNK), jnp.int32),
        pltpu.VMEM((CHUNK, 2 * D), jnp.float32),
        pltpu.VMEM((CHUNK, 2 * D), jnp.float32),
        pltpu.SemaphoreType.DMA,
        pltpu.SemaphoreType.DMA,
    ],
)
def _sc_hash_gather(prev_hbm, cur_hbm, table_hbm, emb_hbm,
                    prev_v, cur_v, bid_v, gbuf_a, gbuf_b, sem_a, sem_b):
    w = lax.axis_index("s") * 2 + lax.axis_index("c")
    pltpu.sync_copy(prev_hbm.at[w], prev_v)
    pltpu.sync_copy(cur_hbm.at[w], cur_v)

    def hash_body(j, carry):
        for v in range(CHUNK // LANES):
            off = v * LANES
            p = prev_v[j, pl.ds(off, LANES)]
            c = cur_v[j, pl.ds(off, LANES)]
            h = p * 32 + c
            h = jnp.where(h >= VOCAB, h - VOCAB, h)
            pos = lax.iota(jnp.int32, LANES) + (j * CHUNK + off)
            h = jnp.where(pos % S == 0, 0, h)
            bid_v[j, pl.ds(off, LANES)] = h
        return carry

    lax.fori_loop(0, NCH, hash_body, 0)

    row0 = w * PER_W

    def writeback(buf, j):
        pltpu.sync_copy(buf, emb_hbm.at[pl.ds(row0 + j * CHUNK, CHUNK)])

    pltpu.async_copy(table_hbm.at[bid_v.at[0]], gbuf_a, sem_a)

    def gather_body(k, carry):
        ja = 2 * k
        jb = 2 * k + 1
        pltpu.async_copy(table_hbm.at[bid_v.at[jb]], gbuf_b, sem_b)
        pltpu.make_async_copy(table_hbm.at[bid_v.at[ja]], gbuf_a, sem_a).wait()
        writeback(gbuf_a, ja)

        @pl.when(ja + 2 < NCH)
        def _():
            pltpu.async_copy(table_hbm.at[bid_v.at[ja + 2]], gbuf_a, sem_a)

        pltpu.make_async_copy(table_hbm.at[bid_v.at[jb]], gbuf_b, sem_b).wait()
        writeback(gbuf_b, jb)
        return carry

    lax.fori_loop(0, NCH // 2, gather_body, 0)


TP_CT = 4096


def _tp_body(tt_ref, out_ref):
    x = tt_ref[...]
    eye = (lax.broadcasted_iota(jnp.int32, (D, D), 0)
           == lax.broadcasted_iota(jnp.int32, (D, D), 1)).astype(jnp.float32)
    xt = lax.dot_general(x, eye, (((0,), (0,)), ((), ())),
                         preferred_element_type=jnp.float32)
    out_ref[...] = jnp.concatenate(
        [xt, jnp.zeros((TP_CT, D), jnp.float32)], axis=1)


def _tc_transpose_pad(table_t):
    grid = (VOCAB + TP_CT - 1) // TP_CT
    return pl.pallas_call(
        _tp_body,
        grid=(grid,),
        in_specs=[pl.BlockSpec((D, TP_CT), lambda i: (0, i))],
        out_specs=pl.BlockSpec((TP_CT, 2 * D), lambda i: (i, 0)),
        out_shape=jax.ShapeDtypeStruct((VOCAB, 2 * D), jnp.float32),
    )(table_t)


def _mm_body(emb_ref, wt_ref, out_ref):
    out_ref[...] = jnp.dot(emb_ref[:, :D], wt_ref[...],
                           preferred_element_type=jnp.float32)


def _tc_matmul(emb_pad, wt):
    RB = 4096
    return pl.pallas_call(
        _mm_body,
        grid=(N // RB,),
        in_specs=[
            pl.BlockSpec((RB, 2 * D), lambda i: (i, 0)),
            pl.BlockSpec((D, MD), lambda i: (0, 0)),
        ],
        out_specs=pl.BlockSpec((RB, MD), lambda i: (i, 0)),
        out_shape=jax.ShapeDtypeStruct((N, MD), jnp.float32),
    )(emb_pad, wt)


def kernel(input_ids, embed_table, W):
    ids = input_ids.reshape(-1).astype(jnp.int32)
    prev = jnp.concatenate([jnp.zeros((1,), jnp.int32), ids[:-1]])
    table_pad = _tc_transpose_pad(embed_table.T)
    emb_pad = _sc_hash_gather(
        prev.reshape(NW, NCH, CHUNK),
        ids.reshape(NW, NCH, CHUNK),
        table_pad,
    )
    out = _tc_matmul(emb_pad, W.T)
    return out.reshape(B, S, MD)


# speedup vs baseline: 1.0089x; 1.0089x over previous
import functools

import jax
import jax.numpy as jnp
from jax import lax
from jax.experimental import pallas as pl
from jax.experimental.pallas import tpu as pltpu
from jax.experimental.pallas import tpu_sc as plsc

VOCAB = 1_000_000
B = 1024
S = 200
D = 64
MD = 1024
N = B * S
K = 4
NK = N // K
NW = 32
PER_W = NK // NW
CHUNK = 80
NCH = PER_W // CHUNK
LANES = 16

_mesh = plsc.VectorSubcoreMesh(core_axis_name="c", subcore_axis_name="s")


@functools.partial(
    pl.kernel,
    mesh=_mesh,
    compiler_params=pltpu.CompilerParams(use_tc_tiling_on_sc=True),
    out_type=jax.ShapeDtypeStruct((NK, 2 * D), jnp.float32),
    scratch_types=[
        pltpu.VMEM((NCH, CHUNK), jnp.int32),
        pltpu.VMEM((NCH, CHUNK), jnp.int32),
        pltpu.VMEM((NCH, CHUNK), jnp.int32),
        pltpu.VMEM((CHUNK, 2 * D), jnp.float32),
        pltpu.VMEM((CHUNK, 2 * D), jnp.float32),
        pltpu.SemaphoreType.DMA,
        pltpu.SemaphoreType.DMA,
    ],
)
def _sc_hash_gather(prev_hbm, cur_hbm, table_hbm, emb_hbm,
                    prev_v, cur_v, bid_v, gbuf_a, gbuf_b, sem_a, sem_b):
    w = lax.axis_index("s") * 2 + lax.axis_index("c")
    pltpu.sync_copy(prev_hbm.at[w], prev_v)
    pltpu.sync_copy(cur_hbm.at[w], cur_v)

    def hash_body(j, carry):
        for v in range(CHUNK // LANES):
            off = v * LANES
            p = prev_v[j, pl.ds(off, LANES)]
            c = cur_v[j, pl.ds(off, LANES)]
            h = p * 32 + c
            h = jnp.where(h >= VOCAB, h - VOCAB, h)
            pos = lax.iota(jnp.int32, LANES) + (j * CHUNK + off)
            h = jnp.where(pos % S == 0, 0, h)
            bid_v[j, pl.ds(off, LANES)] = h
        return carry

    lax.fori_loop(0, NCH, hash_body, 0)

    row0 = w * PER_W

    def writeback(buf, j):
        pltpu.sync_copy(buf, emb_hbm.at[pl.ds(row0 + j * CHUNK, CHUNK)])

    pltpu.async_copy(table_hbm.at[bid_v.at[0]], gbuf_a, sem_a)

    def gather_body(k, carry):
        ja = 2 * k
        jb = 2 * k + 1
        pltpu.async_copy(table_hbm.at[bid_v.at[jb]], gbuf_b, sem_b)
        pltpu.make_async_copy(table_hbm.at[bid_v.at[ja]], gbuf_a, sem_a).wait()
        writeback(gbuf_a, ja)

        @pl.when(ja + 2 < NCH)
        def _():
            pltpu.async_copy(table_hbm.at[bid_v.at[ja + 2]], gbuf_a, sem_a)

        pltpu.make_async_copy(table_hbm.at[bid_v.at[jb]], gbuf_b, sem_b).wait()
        writeback(gbuf_b, jb)
        return carry

    lax.fori_loop(0, NCH // 2, gather_body, 0)


TP_CT = 4096


def _tp_body(tt_ref, out_ref):
    x = tt_ref[...]
    eye = (lax.broadcasted_iota(jnp.int32, (D, D), 0)
           == lax.broadcasted_iota(jnp.int32, (D, D), 1)).astype(jnp.float32)
    xt = lax.dot_general(x, eye, (((0,), (0,)), ((), ())),
                         preferred_element_type=jnp.float32)
    out_ref[...] = jnp.concatenate(
        [xt, jnp.zeros((TP_CT, D), jnp.float32)], axis=1)


def _tc_transpose_pad(table_t):
    grid = (VOCAB + TP_CT - 1) // TP_CT
    return pl.pallas_call(
        _tp_body,
        grid=(grid,),
        in_specs=[pl.BlockSpec((D, TP_CT), lambda i: (0, i))],
        out_specs=pl.BlockSpec((TP_CT, 2 * D), lambda i: (i, 0)),
        out_shape=jax.ShapeDtypeStruct((VOCAB, 2 * D), jnp.float32),
    )(table_t)


RB = 2048


def _mm_body(emb_ref, wt_ref, out_ref):
    out_ref[...] = jnp.dot(emb_ref[:, :D], wt_ref[...],
                           preferred_element_type=jnp.float32)


def _mm_body_acc(prev_ref, emb_ref, wt_ref, out_ref):
    del prev_ref
    out_ref[...] = jnp.dot(emb_ref[:, :D], wt_ref[...],
                           preferred_element_type=jnp.float32)


def _tc_matmul_first(emb_pad, wt):
    return pl.pallas_call(
        _mm_body,
        grid=(NK // RB,),
        in_specs=[
            pl.BlockSpec((RB, 2 * D), lambda i: (i, 0)),
            pl.BlockSpec((D, MD), lambda i: (0, 0)),
        ],
        out_specs=pl.BlockSpec((RB, MD), lambda i: (i, 0)),
        out_shape=jax.ShapeDtypeStruct((N, MD), jnp.float32),
    )(emb_pad, wt)


def _tc_matmul_stage(out_prev, emb_pad, wt, k):
    base = k * (NK // RB)
    return pl.pallas_call(
        _mm_body_acc,
        grid=(NK // RB,),
        in_specs=[
            pl.BlockSpec(memory_space=pl.ANY),
            pl.BlockSpec((RB, 2 * D), lambda i: (i, 0)),
            pl.BlockSpec((D, MD), lambda i: (0, 0)),
        ],
        out_specs=pl.BlockSpec((RB, MD), lambda i, _b=base: (_b + i, 0)),
        out_shape=jax.ShapeDtypeStruct((N, MD), jnp.float32),
        input_output_aliases={0: 0},
    )(out_prev, emb_pad, wt)


def kernel(input_ids, embed_table, W):
    ids = input_ids.reshape(-1).astype(jnp.int32)
    prev = jnp.concatenate([jnp.zeros((1,), jnp.int32), ids[:-1]])
    table_pad = _tc_transpose_pad(embed_table.T)
    wt = W.T
    embs = []
    for k in range(K):
        sl = slice(k * NK, (k + 1) * NK)
        embs.append(_sc_hash_gather(
            prev[sl].reshape(NW, NCH, CHUNK),
            ids[sl].reshape(NW, NCH, CHUNK),
            table_pad,
        ))
    out = _tc_matmul_first(embs[0], wt)
    for k in range(1, K):
        out = _tc_matmul_stage(out, embs[k], wt, k)
    return out.reshape(B, S, MD)


# speedup vs baseline: 1.0241x; 1.0150x over previous
import functools

import jax
import jax.numpy as jnp
from jax import lax
from jax.experimental import pallas as pl
from jax.experimental.pallas import tpu as pltpu
from jax.experimental.pallas import tpu_sc as plsc

VOCAB = 1_000_000
B = 1024
S = 200
D = 64
MD = 1024
N = B * S
K = 4
NK = N // K
NW = 32
PER_W = NK // NW
CHUNK = 80
NCH = PER_W // CHUNK
LANES = 16

_mesh = plsc.VectorSubcoreMesh(core_axis_name="c", subcore_axis_name="s")


@functools.partial(
    pl.kernel,
    mesh=_mesh,
    compiler_params=pltpu.CompilerParams(use_tc_tiling_on_sc=True),
    out_type=jax.ShapeDtypeStruct((NK, 2 * D), jnp.float32),
    scratch_types=[
        pltpu.VMEM((NCH, CHUNK), jnp.int32),
        pltpu.VMEM((NCH, CHUNK), jnp.int32),
        pltpu.VMEM((NCH, CHUNK), jnp.int32),
        pltpu.VMEM((CHUNK, 2 * D), jnp.float32),
        pltpu.VMEM((CHUNK, 2 * D), jnp.float32),
        pltpu.SemaphoreType.DMA,
        pltpu.SemaphoreType.DMA,
    ],
)
def _sc_hash_gather(prev_hbm, cur_hbm, table_hbm, emb_hbm,
                    prev_v, cur_v, bid_v, gbuf_a, gbuf_b, sem_a, sem_b):
    w = lax.axis_index("s") * 2 + lax.axis_index("c")
    pltpu.sync_copy(prev_hbm.at[w], prev_v)
    pltpu.sync_copy(cur_hbm.at[w], cur_v)

    def hash_body(j, carry):
        for v in range(CHUNK // LANES):
            off = v * LANES
            p = prev_v[j, pl.ds(off, LANES)]
            c = cur_v[j, pl.ds(off, LANES)]
            h = p * 32 + c
            h = jnp.where(h >= VOCAB, h - VOCAB, h)
            pos = lax.iota(jnp.int32, LANES) + (j * CHUNK + off)
            h = jnp.where(pos % S == 0, 0, h)
            bid_v[j, pl.ds(off, LANES)] = h
        return carry

    lax.fori_loop(0, NCH, hash_body, 0)

    row0 = w * PER_W

    def writeback(buf, j):
        pltpu.sync_copy(buf, emb_hbm.at[pl.ds(row0 + j * CHUNK, CHUNK)])

    pltpu.async_copy(table_hbm.at[bid_v.at[0]], gbuf_a, sem_a)

    def gather_body(k, carry):
        ja = 2 * k
        jb = 2 * k + 1
        pltpu.async_copy(table_hbm.at[bid_v.at[jb]], gbuf_b, sem_b)
        pltpu.make_async_copy(table_hbm.at[bid_v.at[ja]], gbuf_a, sem_a).wait()
        writeback(gbuf_a, ja)

        @pl.when(ja + 2 < NCH)
        def _():
            pltpu.async_copy(table_hbm.at[bid_v.at[ja + 2]], gbuf_a, sem_a)

        pltpu.make_async_copy(table_hbm.at[bid_v.at[jb]], gbuf_b, sem_b).wait()
        writeback(gbuf_b, jb)
        return carry

    lax.fori_loop(0, NCH // 2, gather_body, 0)


TP_CT = 4096


def _tp_body(tt_ref, out_ref):
    x = tt_ref[...]
    xt = x.T
    out_ref[...] = jnp.concatenate(
        [xt, jnp.zeros((TP_CT, D), jnp.float32)], axis=1)


def _tc_transpose_pad(table_t):
    grid = (VOCAB + TP_CT - 1) // TP_CT
    return pl.pallas_call(
        _tp_body,
        grid=(grid,),
        in_specs=[pl.BlockSpec((D, TP_CT), lambda i: (0, i))],
        out_specs=pl.BlockSpec((TP_CT, 2 * D), lambda i: (i, 0)),
        out_shape=jax.ShapeDtypeStruct((VOCAB, 2 * D), jnp.float32),
    )(table_t)


RB = 2048


def _mm_body(emb_ref, wt_ref, out_ref):
    out_ref[...] = jnp.dot(emb_ref[:, :D], wt_ref[...],
                           preferred_element_type=jnp.float32)


def _mm_body_acc(prev_ref, emb_ref, wt_ref, out_ref):
    del prev_ref
    out_ref[...] = jnp.dot(emb_ref[:, :D], wt_ref[...],
                           preferred_element_type=jnp.float32)


def _tc_matmul_first(emb_pad, wt):
    return pl.pallas_call(
        _mm_body,
        grid=(NK // RB,),
        in_specs=[
            pl.BlockSpec((RB, 2 * D), lambda i: (i, 0)),
            pl.BlockSpec((D, MD), lambda i: (0, 0)),
        ],
        out_specs=pl.BlockSpec((RB, MD), lambda i: (i, 0)),
        out_shape=jax.ShapeDtypeStruct((N, MD), jnp.float32),
    )(emb_pad, wt)


def _tc_matmul_stage(out_prev, emb_pad, wt, k):
    base = k * (NK // RB)
    return pl.pallas_call(
        _mm_body_acc,
        grid=(NK // RB,),
        in_specs=[
            pl.BlockSpec(memory_space=pl.ANY),
            pl.BlockSpec((RB, 2 * D), lambda i: (i, 0)),
            pl.BlockSpec((D, MD), lambda i: (0, 0)),
        ],
        out_specs=pl.BlockSpec((RB, MD), lambda i, _b=base: (_b + i, 0)),
        out_shape=jax.ShapeDtypeStruct((N, MD), jnp.float32),
        input_output_aliases={0: 0},
    )(out_prev, emb_pad, wt)


def kernel(input_ids, embed_table, W):
    ids = input_ids.reshape(-1).astype(jnp.int32)
    prev = jnp.concatenate([jnp.zeros((1,), jnp.int32), ids[:-1]])
    table_pad = _tc_transpose_pad(embed_table.T)
    wt = W.T
    embs = []
    for k in range(K):
        sl = slice(k * NK, (k + 1) * NK)
        embs.append(_sc_hash_gather(
            prev[sl].reshape(NW, NCH, CHUNK),
            ids[sl].reshape(NW, NCH, CHUNK),
            table_pad,
        ))
    out = _tc_matmul_first(embs[0], wt)
    for k in range(1, K):
        out = _tc_matmul_stage(out, embs[k], wt, k)
    return out.reshape(B, S, MD)


# speedup vs baseline: 1.1280x; 1.1014x over previous
import functools

import jax
import jax.numpy as jnp
from jax import lax
from jax.experimental import pallas as pl
from jax.experimental.pallas import tpu as pltpu
from jax.experimental.pallas import tpu_sc as plsc

VOCAB = 1_000_000
B = 1024
S = 200
D = 64
MD = 1024
N = B * S
K = 4
NK = N // K
NW = 32
PER_W = NK // NW
CHUNK = 80
NCH = PER_W // CHUNK
LANES = 16

_mesh = plsc.VectorSubcoreMesh(core_axis_name="c", subcore_axis_name="s")


@functools.partial(
    pl.kernel,
    mesh=_mesh,
    compiler_params=pltpu.CompilerParams(use_tc_tiling_on_sc=True),
    out_type=jax.ShapeDtypeStruct((NK, 2 * D), jnp.float32),
    scratch_types=[
        pltpu.VMEM((NCH, CHUNK), jnp.int32),
        pltpu.VMEM((NCH, CHUNK), jnp.int32),
        pltpu.VMEM((NCH, CHUNK), jnp.int32),
        pltpu.VMEM((CHUNK, 2 * D), jnp.float32),
        pltpu.VMEM((CHUNK, 2 * D), jnp.float32),
        pltpu.SemaphoreType.DMA,
        pltpu.SemaphoreType.DMA,
    ],
)
def _sc_hash_gather(prev_hbm, cur_hbm, table_hbm, emb_hbm,
                    prev_v, cur_v, bid_v, gbuf_a, gbuf_b, sem_a, sem_b):
    w = lax.axis_index("s") * 2 + lax.axis_index("c")
    pltpu.sync_copy(prev_hbm.at[w], prev_v)
    pltpu.sync_copy(cur_hbm.at[w], cur_v)

    def hash_body(j, carry):
        for v in range(CHUNK // LANES):
            off = v * LANES
            p = prev_v[j, pl.ds(off, LANES)]
            c = cur_v[j, pl.ds(off, LANES)]
            h = p * 32 + c
            h = jnp.where(h >= VOCAB, h - VOCAB, h)
            pos = lax.iota(jnp.int32, LANES) + (j * CHUNK + off)
            h = jnp.where(pos % S == 0, 0, h)
            bid_v[j, pl.ds(off, LANES)] = h
        return carry

    lax.fori_loop(0, NCH, hash_body, 0)

    row0 = w * PER_W

    def writeback(buf, j):
        pltpu.sync_copy(buf, emb_hbm.at[pl.ds(row0 + j * CHUNK, CHUNK)])

    pltpu.async_copy(table_hbm.at[bid_v.at[0]], gbuf_a, sem_a)

    def gather_body(k, carry):
        ja = 2 * k
        jb = 2 * k + 1
        pltpu.async_copy(table_hbm.at[bid_v.at[jb]], gbuf_b, sem_b)
        pltpu.make_async_copy(table_hbm.at[bid_v.at[ja]], gbuf_a, sem_a).wait()
        writeback(gbuf_a, ja)

        @pl.when(ja + 2 < NCH)
        def _():
            pltpu.async_copy(table_hbm.at[bid_v.at[ja + 2]], gbuf_a, sem_a)

        pltpu.make_async_copy(table_hbm.at[bid_v.at[jb]], gbuf_b, sem_b).wait()
        writeback(gbuf_b, jb)
        return carry

    lax.fori_loop(0, NCH // 2, gather_body, 0)


TP_CT = 8192


def _tp_body(tt_ref, out_ref):
    x = tt_ref[...]
    xt = x.T
    out_ref[...] = jnp.concatenate(
        [xt, jnp.zeros((TP_CT, D), jnp.float32)], axis=1)


def _tc_transpose_pad(table_t):
    grid = (VOCAB + TP_CT - 1) // TP_CT
    return pl.pallas_call(
        _tp_body,
        grid=(grid,),
        in_specs=[pl.BlockSpec((D, TP_CT), lambda i: (0, i))],
        out_specs=pl.BlockSpec((TP_CT, 2 * D), lambda i: (i, 0)),
        out_shape=jax.ShapeDtypeStruct((VOCAB, 2 * D), jnp.float32),
    )(table_t)


RB = 2048


def _mm_body(emb_ref, wt_ref, out_ref):
    out_ref[...] = jnp.dot(emb_ref[:, :D], wt_ref[...],
                           preferred_element_type=jnp.float32)


def _mm_body_acc(prev_ref, emb_ref, wt_ref, out_ref):
    del prev_ref
    out_ref[...] = jnp.dot(emb_ref[:, :D], wt_ref[...],
                           preferred_element_type=jnp.float32)


def _tc_matmul_first(emb_pad, wt):
    return pl.pallas_call(
        _mm_body,
        grid=(NK // RB,),
        in_specs=[
            pl.BlockSpec((RB, 2 * D), lambda i: (i, 0)),
            pl.BlockSpec((D, MD), lambda i: (0, 0)),
        ],
        out_specs=pl.BlockSpec((RB, MD), lambda i: (i, 0)),
        out_shape=jax.ShapeDtypeStruct((N, MD), jnp.float32),
    )(emb_pad, wt)


def _tc_matmul_stage(out_prev, emb_pad, wt, k):
    base = k * (NK // RB)
    return pl.pallas_call(
        _mm_body_acc,
        grid=(NK // RB,),
        in_specs=[
            pl.BlockSpec(memory_space=pl.ANY),
            pl.BlockSpec((RB, 2 * D), lambda i: (i, 0)),
            pl.BlockSpec((D, MD), lambda i: (0, 0)),
        ],
        out_specs=pl.BlockSpec((RB, MD), lambda i, _b=base: (_b + i, 0)),
        out_shape=jax.ShapeDtypeStruct((N, MD), jnp.float32),
        input_output_aliases={0: 0},
    )(out_prev, emb_pad, wt)


def kernel(input_ids, embed_table, W):
    ids = input_ids.reshape(-1).astype(jnp.int32)
    prev = jnp.concatenate([jnp.zeros((1,), jnp.int32), ids[:-1]])
    table_pad = _tc_transpose_pad(embed_table.T)
    wt = W.T
    embs = []
    for k in range(K):
        sl = slice(k * NK, (k + 1) * NK)
        embs.append(_sc_hash_gather(
            prev[sl].reshape(NW, NCH, CHUNK),
            ids[sl].reshape(NW, NCH, CHUNK),
            table_pad,
        ))
    out = _tc_matmul_first(embs[0], wt)
    for k in range(1, K):
        out = _tc_matmul_stage(out, embs[k], wt, k)
    return out.reshape(B, S, MD)


# speedup vs baseline: 1.1609x; 1.0292x over previous
import functools

import jax
import jax.numpy as jnp
from jax import lax
from jax.experimental import pallas as pl
from jax.experimental.pallas import tpu as pltpu
from jax.experimental.pallas import tpu_sc as plsc

VOCAB = 1_000_000
B = 1024
S = 200
D = 64
MD = 1024
N = B * S
K = 4
NK = N // K
NW = 32
PER_W = NK // NW
CHUNK = 80
NCH = PER_W // CHUNK
LANES = 16

_mesh = plsc.VectorSubcoreMesh(core_axis_name="c", subcore_axis_name="s")


@functools.partial(
    pl.kernel,
    mesh=_mesh,
    compiler_params=pltpu.CompilerParams(use_tc_tiling_on_sc=True),
    out_type=jax.ShapeDtypeStruct((NK, 2 * D), jnp.float32),
    scratch_types=[
        pltpu.VMEM((NCH, CHUNK), jnp.int32),
        pltpu.VMEM((NCH, CHUNK), jnp.int32),
        pltpu.VMEM((NCH, CHUNK), jnp.int32),
        pltpu.VMEM((CHUNK, 2 * D), jnp.float32),
        pltpu.VMEM((CHUNK, 2 * D), jnp.float32),
        pltpu.SemaphoreType.DMA,
        pltpu.SemaphoreType.DMA,
    ],
)
def _sc_hash_gather(prev_hbm, cur_hbm, table_hbm, emb_hbm,
                    prev_v, cur_v, bid_v, gbuf_a, gbuf_b, sem_a, sem_b):
    w = lax.axis_index("s") * 2 + lax.axis_index("c")
    pltpu.sync_copy(prev_hbm.at[w], prev_v)
    pltpu.sync_copy(cur_hbm.at[w], cur_v)

    def hash_body(j, carry):
        for v in range(CHUNK // LANES):
            off = v * LANES
            p = prev_v[j, pl.ds(off, LANES)]
            c = cur_v[j, pl.ds(off, LANES)]
            h = p * 32 + c
            h = jnp.where(h >= VOCAB, h - VOCAB, h)
            pos = lax.iota(jnp.int32, LANES) + (j * CHUNK + off)
            h = jnp.where(pos % S == 0, 0, h)
            bid_v[j, pl.ds(off, LANES)] = h
        return carry

    lax.fori_loop(0, NCH, hash_body, 0)

    row0 = w * PER_W

    def writeback(buf, j):
        pltpu.sync_copy(buf, emb_hbm.at[pl.ds(row0 + j * CHUNK, CHUNK)])

    pltpu.async_copy(table_hbm.at[bid_v.at[0]], gbuf_a, sem_a)

    def gather_body(k, carry):
        ja = 2 * k
        jb = 2 * k + 1
        pltpu.async_copy(table_hbm.at[bid_v.at[jb]], gbuf_b, sem_b)
        pltpu.make_async_copy(table_hbm.at[bid_v.at[ja]], gbuf_a, sem_a).wait()
        writeback(gbuf_a, ja)

        @pl.when(ja + 2 < NCH)
        def _():
            pltpu.async_copy(table_hbm.at[bid_v.at[ja + 2]], gbuf_a, sem_a)

        pltpu.make_async_copy(table_hbm.at[bid_v.at[jb]], gbuf_b, sem_b).wait()
        writeback(gbuf_b, jb)
        return carry

    lax.fori_loop(0, NCH // 2, gather_body, 0)


TP_CT = 16384


def _tp_body(tt_ref, out_ref):
    x = tt_ref[...]
    xt = x.T
    out_ref[...] = jnp.concatenate(
        [xt, jnp.zeros((TP_CT, D), jnp.float32)], axis=1)


def _tc_transpose_pad(table_t):
    grid = (VOCAB + TP_CT - 1) // TP_CT
    return pl.pallas_call(
        _tp_body,
        grid=(grid,),
        in_specs=[pl.BlockSpec((D, TP_CT), lambda i: (0, i))],
        out_specs=pl.BlockSpec((TP_CT, 2 * D), lambda i: (i, 0)),
        out_shape=jax.ShapeDtypeStruct((VOCAB, 2 * D), jnp.float32),
    )(table_t)


RB = 2048


def _mm_body(emb_ref, wt_ref, out_ref):
    out_ref[...] = jnp.dot(emb_ref[:, :D], wt_ref[...],
                           preferred_element_type=jnp.float32)


def _mm_body_acc(prev_ref, emb_ref, wt_ref, out_ref):
    del prev_ref
    out_ref[...] = jnp.dot(emb_ref[:, :D], wt_ref[...],
                           preferred_element_type=jnp.float32)


def _tc_matmul_first(emb_pad, wt):
    return pl.pallas_call(
        _mm_body,
        grid=(NK // RB,),
        in_specs=[
            pl.BlockSpec((RB, 2 * D), lambda i: (i, 0)),
            pl.BlockSpec((D, MD), lambda i: (0, 0)),
        ],
        out_specs=pl.BlockSpec((RB, MD), lambda i: (i, 0)),
        out_shape=jax.ShapeDtypeStruct((N, MD), jnp.float32),
    )(emb_pad, wt)


def _tc_matmul_stage(out_prev, emb_pad, wt, k):
    base = k * (NK // RB)
    return pl.pallas_call(
        _mm_body_acc,
        grid=(NK // RB,),
        in_specs=[
            pl.BlockSpec(memory_space=pl.ANY),
            pl.BlockSpec((RB, 2 * D), lambda i: (i, 0)),
            pl.BlockSpec((D, MD), lambda i: (0, 0)),
        ],
        out_specs=pl.BlockSpec((RB, MD), lambda i, _b=base: (_b + i, 0)),
        out_shape=jax.ShapeDtypeStruct((N, MD), jnp.float32),
        input_output_aliases={0: 0},
    )(out_prev, emb_pad, wt)


def kernel(input_ids, embed_table, W):
    ids = input_ids.reshape(-1).astype(jnp.int32)
    prev = jnp.concatenate([jnp.zeros((1,), jnp.int32), ids[:-1]])
    table_pad = _tc_transpose_pad(embed_table.T)
    wt = W.T
    embs = []
    for k in range(K):
        sl = slice(k * NK, (k + 1) * NK)
        embs.append(_sc_hash_gather(
            prev[sl].reshape(NW, NCH, CHUNK),
            ids[sl].reshape(NW, NCH, CHUNK),
            table_pad,
        ))
    out = _tc_matmul_first(embs[0], wt)
    for k in range(1, K):
        out = _tc_matmul_stage(out, embs[k], wt, k)
    return out.reshape(B, S, MD)


# speedup vs baseline: 1.1782x; 1.0149x over previous
import functools

import jax
import jax.numpy as jnp
from jax import lax
from jax.experimental import pallas as pl
from jax.experimental.pallas import tpu as pltpu
from jax.experimental.pallas import tpu_sc as plsc

VOCAB = 1_000_000
B = 1024
S = 200
D = 64
MD = 1024
N = B * S
K = 4
NK = N // K
NW = 32
PER_W = NK // NW
CHUNK = 80
NCH = PER_W // CHUNK
LANES = 16

_mesh = plsc.VectorSubcoreMesh(core_axis_name="c", subcore_axis_name="s")


@functools.partial(
    pl.kernel,
    mesh=_mesh,
    compiler_params=pltpu.CompilerParams(use_tc_tiling_on_sc=True),
    out_type=jax.ShapeDtypeStruct((NK, 2 * D), jnp.float32),
    scratch_types=[
        pltpu.VMEM((NCH, CHUNK), jnp.int32),
        pltpu.VMEM((NCH, CHUNK), jnp.int32),
        pltpu.VMEM((NCH, CHUNK), jnp.int32),
        pltpu.VMEM((CHUNK, 2 * D), jnp.float32),
        pltpu.VMEM((CHUNK, 2 * D), jnp.float32),
        pltpu.SemaphoreType.DMA,
        pltpu.SemaphoreType.DMA,
    ],
)
def _sc_hash_gather(prev_hbm, cur_hbm, table_hbm, emb_hbm,
                    prev_v, cur_v, bid_v, gbuf_a, gbuf_b, sem_a, sem_b):
    w = lax.axis_index("s") * 2 + lax.axis_index("c")
    pltpu.sync_copy(prev_hbm.at[w], prev_v)
    pltpu.sync_copy(cur_hbm.at[w], cur_v)

    def hash_body(j, carry):
        for v in range(CHUNK // LANES):
            off = v * LANES
            p = prev_v[j, pl.ds(off, LANES)]
            c = cur_v[j, pl.ds(off, LANES)]
            h = p * 32 + c
            h = jnp.where(h >= VOCAB, h - VOCAB, h)
            pos = lax.iota(jnp.int32, LANES) + (j * CHUNK + off)
            h = jnp.where(pos % S == 0, 0, h)
            bid_v[j, pl.ds(off, LANES)] = h
        return carry

    lax.fori_loop(0, NCH, hash_body, 0)

    row0 = w * PER_W

    def writeback(buf, j):
        pltpu.sync_copy(buf, emb_hbm.at[pl.ds(row0 + j * CHUNK, CHUNK)])

    pltpu.async_copy(table_hbm.at[bid_v.at[0]], gbuf_a, sem_a)

    def gather_body(k, carry):
        ja = 2 * k
        jb = 2 * k + 1
        pltpu.async_copy(table_hbm.at[bid_v.at[jb]], gbuf_b, sem_b)
        pltpu.make_async_copy(table_hbm.at[bid_v.at[ja]], gbuf_a, sem_a).wait()
        writeback(gbuf_a, ja)

        @pl.when(ja + 2 < NCH)
        def _():
            pltpu.async_copy(table_hbm.at[bid_v.at[ja + 2]], gbuf_a, sem_a)

        pltpu.make_async_copy(table_hbm.at[bid_v.at[jb]], gbuf_b, sem_b).wait()
        writeback(gbuf_b, jb)
        return carry

    lax.fori_loop(0, NCH // 2, gather_body, 0)


TP_CT = 32768


def _tp_body(tt_ref, out_ref):
    x = tt_ref[...]
    xt = x.T
    out_ref[...] = jnp.concatenate(
        [xt, jnp.zeros((TP_CT, D), jnp.float32)], axis=1)


def _tc_transpose_pad(table_t):
    grid = (VOCAB + TP_CT - 1) // TP_CT
    return pl.pallas_call(
        _tp_body,
        grid=(grid,),
        in_specs=[pl.BlockSpec((D, TP_CT), lambda i: (0, i))],
        out_specs=pl.BlockSpec((TP_CT, 2 * D), lambda i: (i, 0)),
        out_shape=jax.ShapeDtypeStruct((VOCAB, 2 * D), jnp.float32),
    )(table_t)


RB = 3200


def _mm_body(emb_ref, wt_ref, out_ref):
    out_ref[...] = jnp.dot(emb_ref[:, :D], wt_ref[...],
                           preferred_element_type=jnp.float32)


def _mm_body_acc(prev_ref, emb_ref, wt_ref, out_ref):
    del prev_ref
    out_ref[...] = jnp.dot(emb_ref[:, :D], wt_ref[...],
                           preferred_element_type=jnp.float32)


def _tc_matmul_first(emb_pad, wt):
    return pl.pallas_call(
        _mm_body,
        grid=(NK // RB,),
        in_specs=[
            pl.BlockSpec((RB, 2 * D), lambda i: (i, 0)),
            pl.BlockSpec((D, MD), lambda i: (0, 0)),
        ],
        out_specs=pl.BlockSpec((RB, MD), lambda i: (i, 0)),
        out_shape=jax.ShapeDtypeStruct((N, MD), jnp.float32),
    )(emb_pad, wt)


def _tc_matmul_stage(out_prev, emb_pad, wt, k):
    base = k * (NK // RB)
    return pl.pallas_call(
        _mm_body_acc,
        grid=(NK // RB,),
        in_specs=[
            pl.BlockSpec(memory_space=pl.ANY),
            pl.BlockSpec((RB, 2 * D), lambda i: (i, 0)),
            pl.BlockSpec((D, MD), lambda i: (0, 0)),
        ],
        out_specs=pl.BlockSpec((RB, MD), lambda i, _b=base: (_b + i, 0)),
        out_shape=jax.ShapeDtypeStruct((N, MD), jnp.float32),
        input_output_aliases={0: 0},
    )(out_prev, emb_pad, wt)


def kernel(input_ids, embed_table, W):
    ids = input_ids.reshape(-1).astype(jnp.int32)
    prev = jnp.concatenate([jnp.zeros((1,), jnp.int32), ids[:-1]])
    table_pad = _tc_transpose_pad(embed_table.T)
    wt = W.T
    embs = []
    for k in range(K):
        sl = slice(k * NK, (k + 1) * NK)
        embs.append(_sc_hash_gather(
            prev[sl].reshape(NW, NCH, CHUNK),
            ids[sl].reshape(NW, NCH, CHUNK),
            table_pad,
        ))
    out = _tc_matmul_first(embs[0], wt)
    for k in range(1, K):
        out = _tc_matmul_stage(out, embs[k], wt, k)
    return out.reshape(B, S, MD)
